# Initial kernel scaffold; baseline (speedup 1.0000x reference)
import jax
import jax.numpy as jnp
from jax.experimental import pallas as pl


def kernel(texts, embedding_weight):
    raise NotImplementedError("write your pallas kernel here")


# speedup vs baseline: 5.0702x; 5.0702x over previous
import functools

import jax
import jax.numpy as jnp
from jax import lax
from jax.experimental import pallas as pl
from jax.experimental.pallas import tpu as pltpu
from jax.experimental.pallas import tpu_sc as plsc

B = 4096
L = 200
VOCAB = 50257
DIM = 32
T = B * L

NC = 2
NS = 16
NW = NC * NS
TOK_PER_W = T // NW
CHUNK = 1280
NCHUNK = TOK_PER_W // CHUNK


def _binarize_body(w_ref, o_ref):
    o_ref[...] = (w_ref[...] >= 0.0).astype(jnp.float32)


def _binarize_table(w):
    rb = 8192
    grid = (VOCAB + rb - 1) // rb
    return pl.pallas_call(
        _binarize_body,
        out_shape=jax.ShapeDtypeStruct((VOCAB, DIM), jnp.float32),
        grid=(grid,),
        in_specs=[pl.BlockSpec((rb, DIM), lambda i: (i, 0))],
        out_specs=pl.BlockSpec((rb, DIM), lambda i: (i, 0)),
    )(w)


_sc_mesh = plsc.VectorSubcoreMesh(
    core_axis_name="c", subcore_axis_name="s", num_cores=NC
)


@functools.partial(
    pl.kernel,
    mesh=_sc_mesh,
    compiler_params=pltpu.CompilerParams(use_tc_tiling_on_sc=False),
    out_type=jax.ShapeDtypeStruct((T, DIM), jnp.float32),
    scratch_types=[
        pltpu.VMEM((TOK_PER_W,), jnp.int32),
        pltpu.VMEM((CHUNK, DIM), jnp.float32),
        pltpu.SemaphoreType.DMA,
    ],
)
def _sc_gather(bin_hbm, idx_hbm, out_hbm, idx_v, rows_v, gsem):
    wid = lax.axis_index("s") * NC + lax.axis_index("c")
    base = wid * TOK_PER_W
    pltpu.sync_copy(idx_hbm.at[pl.ds(base, TOK_PER_W)], idx_v)

    def chunk_body(i, carry):
        idx_sl = idx_v.at[pl.ds(i * CHUNK, CHUNK)]
        pltpu.async_copy(bin_hbm.at[idx_sl], rows_v, gsem).wait()
        pltpu.sync_copy(rows_v, out_hbm.at[pl.ds(base + i * CHUNK, CHUNK)])
        return carry

    lax.fori_loop(0, NCHUNK, chunk_body, 0)


def kernel(texts, embedding_weight):
    bin_table = _binarize_table(embedding_weight)
    idx = texts.reshape(T)
    out = _sc_gather(bin_table, idx)
    return out.reshape(B, L, DIM)


# speedup vs baseline: 7.6468x; 1.5082x over previous
import functools

import jax
import jax.numpy as jnp
from jax import lax
from jax.experimental import pallas as pl
from jax.experimental.pallas import tpu as pltpu
from jax.experimental.pallas import tpu_sc as plsc

B = 4096
L = 200
VOCAB = 50257
DIM = 32
T = B * L

VP = 51200
PACK_C = 2048
NPACK = VP // PACK_C

NC = 2
NS = 16
NW = NC * NS
TOK_PER_W = T // NW

UNPACK_BM = 256


def _pack_body(wt_ref, o_ref):
    x = wt_ref[...]
    sh = lax.broadcasted_iota(jnp.int32, (DIM, PACK_C), 0)
    bits = jnp.sum(jnp.where(x >= 0.0, 1, 0) << sh, axis=0)
    o_ref[...] = bits[None, None, :]


def _pack_table(w):
    wt = jnp.pad(w.T, ((0, 0), (0, VP - VOCAB)))
    pw = pl.pallas_call(
        _pack_body,
        out_shape=jax.ShapeDtypeStruct((NPACK, 1, PACK_C), jnp.int32),
        grid=(NPACK,),
        in_specs=[pl.BlockSpec((DIM, PACK_C), lambda i: (0, i))],
        out_specs=pl.BlockSpec((1, 1, PACK_C), lambda i: (i, 0, 0)),
    )(wt)
    return pw.reshape(VP)


_sc_mesh = plsc.VectorSubcoreMesh(
    core_axis_name="c", subcore_axis_name="s", num_cores=NC
)


@functools.partial(
    pl.kernel,
    mesh=_sc_mesh,
    compiler_params=pltpu.CompilerParams(
        use_tc_tiling_on_sc=False, needs_layout_passes=False
    ),
    out_type=jax.ShapeDtypeStruct((T,), jnp.int32),
    scratch_types=[
        pltpu.VMEM((VP,), jnp.int32),
        pltpu.VMEM((TOK_PER_W,), jnp.int32),
        pltpu.VMEM((TOK_PER_W,), jnp.int32),
    ],
)
def _sc_word_gather(pw_hbm, idx_hbm, out_hbm, tab_v, ids_v, wv):
    wid = lax.axis_index("s") * NC + lax.axis_index("c")
    base = wid * TOK_PER_W
    pltpu.sync_copy(pw_hbm, tab_v)
    pltpu.sync_copy(idx_hbm.at[pl.ds(base, TOK_PER_W)], ids_v)

    def body(i, carry):
        sl = pl.ds(i * 16, 16)
        idx = ids_v[sl]
        wv[sl] = plsc.load_gather(tab_v, [idx])
        return carry

    lax.fori_loop(0, TOK_PER_W // 16, body, 0)
    pltpu.sync_copy(wv, out_hbm.at[pl.ds(base, TOK_PER_W)])


def _unpack_body(w_ref, o_ref):
    w = w_ref[...]
    wr = jnp.broadcast_to(w[:, :, None], (UNPACK_BM, L, DIM))
    wr = wr.reshape(UNPACK_BM, L * DIM)
    d = lax.broadcasted_iota(jnp.int32, (UNPACK_BM, L * DIM), 1) & 31
    o_ref[...] = ((wr >> d) & 1).astype(jnp.float32)


def _unpack(words2d):
    return pl.pallas_call(
        _unpack_body,
        out_shape=jax.ShapeDtypeStruct((B, L * DIM), jnp.float32),
        grid=(B // UNPACK_BM,),
        in_specs=[pl.BlockSpec((UNPACK_BM, L), lambda i: (i, 0))],
        out_specs=pl.BlockSpec((UNPACK_BM, L * DIM), lambda i: (i, 0)),
    )(words2d)


def kernel(texts, embedding_weight):
    pw = _pack_table(embedding_weight)
    idx = texts.reshape(T)
    words = _sc_word_gather(pw, idx)
    out = _unpack(words.reshape(B, L))
    return out.reshape(B, L, DIM)
